# baseline (speedup 1.0000x reference)
import functools

import jax
import jax.numpy as jnp
from jax import lax
from jax.experimental import pallas as pl
from jax.experimental.pallas import tpu as pltpu
from jax.experimental.pallas import tpu_sc as plsc

_NC = 2
_NS = 16
_L = 16
_DW = 16

_INTERPRET = False


def _pre_body(x_ref, wa_ref, wb_ref, wp_ref, wq_ref, bb_ref, bp_ref, bq_ref,
              a_out, b_out, p_out, q_out):
    x = x_ref[...]
    a_out[...] = jnp.dot(x, wa_ref[...], preferred_element_type=jnp.float32)
    b_out[...] = jnp.dot(x, wb_ref[...], preferred_element_type=jnp.float32) + bb_ref[...]
    p_out[...] = jnp.dot(x, wp_ref[...], preferred_element_type=jnp.float32) + bp_ref[...]
    q_out[...] = jnp.dot(x, wq_ref[...], preferred_element_type=jnp.float32) + bq_ref[...]


def _post_body(x_ref, hin0_ref, hin1_ref, hout0_ref, hout1_ref,
               din0_ref, din1_ref, dout0_ref, dout1_ref,
               wga_ref, wgb_ref, bg1_ref, wg2_ref, bg2_ref, out_ref):
    hin = hin0_ref[...] + hin1_ref[...]
    hout = hout0_ref[...] + hout1_ref[...]
    din = jnp.maximum(din0_ref[:, 0:1] + din1_ref[:, 0:1], 1.0)
    dout = jnp.maximum(dout0_ref[:, 0:1] + dout1_ref[:, 0:1], 1.0)
    h_in = hin / din
    h_out = hout / dout
    gh = jnp.maximum(
        jnp.dot(h_in, wga_ref[...], preferred_element_type=jnp.float32)
        + jnp.dot(h_out, wgb_ref[...], preferred_element_type=jnp.float32)
        + bg1_ref[...], 0.0)
    gz = jnp.sum(gh * wg2_ref[...], axis=1, keepdims=True) + bg2_ref[...]
    g = 1.0 / (1.0 + jnp.exp(-gz))
    out_ref[...] = g * h_in + (1.0 - g) * h_out + x_ref[...]


def _make_edge_kernel(n_nodes, n_edges, d, ck):
    nw = _NC * _NS
    e_per_w = n_edges // nw
    e_per_sc = n_edges // _NC
    n_chunks = e_per_w // ck
    groups = ck // _L
    assert e_per_w * nw == n_edges and n_chunks * ck == e_per_w
    assert groups * _L == ck and n_nodes % 8 == 0
    span = 8 * (-(-(n_nodes // 8) // _NS))
    zr = 8
    zit = span // zr
    assert zr * zit == span

    mesh = plsc.VectorSubcoreMesh(core_axis_name="c", subcore_axis_name="s",
                                  num_cores=_NC, num_subcores=_NS)

    def body(src_hbm, dst_hbm, a_hbm, b_hbm, p_hbm, q_hbm, wb2_hbm,
             zrow_hbm, zdeg_hbm, ones_hbm,
             hin_out, hout_out, din_out, dout_out, scores_out,
             h_acc, deg_acc,
             idx_s, idx_d, abuf, bbuf, pbuf, sbuf, ones_v, zbuf, zdeg, w2_v,
             sem_a, sem_b, sem_p):
        c = lax.axis_index("c")
        s = lax.axis_index("s")
        wid = c * _NS + s
        lane = lax.iota(jnp.int32, _L)

        pltpu.sync_copy(wb2_hbm, w2_v)
        pltpu.sync_copy(zrow_hbm, zbuf)
        pltpu.sync_copy(zdeg_hbm, zdeg)
        pltpu.sync_copy(ones_hbm, ones_v)

        r0 = pl.multiple_of(jnp.minimum(s * span, n_nodes - span), 8)

        def clear_acc():
            def zero_step(t, carry):
                off = pl.multiple_of(r0 + t * zr, 8)
                pltpu.sync_copy(zbuf, h_acc.at[pl.ds(off, zr)])
                pltpu.sync_copy(zdeg, deg_acc.at[pl.ds(off, zr)])
                return carry
            lax.fori_loop(0, zit, zero_step, 0)

        clear_acc()
        plsc.subcore_barrier()

        be2v = plsc.load_gather(w2_v, [jnp.full((_L,), d, jnp.int32)])

        def phase1_chunk(i, carry):
            base = pl.multiple_of(wid * e_per_w + i * ck, 8)
            pltpu.sync_copy(src_hbm.at[pl.ds(base, ck)], idx_s)
            pltpu.sync_copy(dst_hbm.at[pl.ds(base, ck)], idx_d)
            cp_a = pltpu.async_copy(a_hbm.at[idx_s], abuf, sem_a)
            cp_b = pltpu.async_copy(b_hbm.at[idx_d], bbuf, sem_b)
            cp_p = pltpu.async_copy(p_hbm.at[idx_s], pbuf, sem_p)
            cp_a.wait()
            cp_b.wait()
            cp_p.wait()

            def group_body(g, carry2):
                e0 = pl.multiple_of(g * _L, _L)
                e_idx = lane + e0
                kv0 = jnp.zeros((_L,), jnp.int32)

                def score_step(k, state):
                    acc, kv = state
                    av = plsc.load_gather(abuf, [e_idx, kv])
                    bv = plsc.load_gather(bbuf, [e_idx, kv])
                    wk = plsc.load_gather(w2_v, [kv])
                    return acc + jnp.maximum(av + bv, 0.0) * wk, kv + 1

                acc, _ = lax.fori_loop(0, d, score_step, (be2v, kv0))
                score = 1.0 / (1.0 + jnp.exp(-acc))
                sbuf[pl.ds(e0, _L)] = score

                def msg_step(k, kv):
                    pv = plsc.load_gather(pbuf, [e_idx, kv])
                    plsc.store_scatter(pbuf, [e_idx, kv], pv * score)
                    return kv + 1

                lax.fori_loop(0, d, msg_step, kv0)
                return carry2

            lax.fori_loop(0, groups, group_body, 0)
            pltpu.sync_copy(sbuf, scores_out.at[pl.ds(base, ck)])
            pltpu.sync_copy(pbuf, h_acc.at[idx_d], add=True)
            pltpu.sync_copy(ones_v, deg_acc.at[idx_d], add=True)
            return carry

        lax.fori_loop(0, n_chunks, phase1_chunk, 0)
        plsc.subcore_barrier()

        out_r0 = pl.multiple_of(c * n_nodes + r0, 8)
        pltpu.sync_copy(h_acc.at[pl.ds(r0, span)],
                        hin_out.at[pl.ds(out_r0, span)])
        pltpu.sync_copy(deg_acc.at[pl.ds(r0, span)],
                        din_out.at[pl.ds(out_r0, span)])
        plsc.subcore_barrier()
        clear_acc()
        plsc.subcore_barrier()

        def phase2_chunk(i, carry):
            base = pl.multiple_of(wid * e_per_w + i * ck, 8)
            pltpu.sync_copy(src_hbm.at[pl.ds(base, ck)], idx_s)
            pltpu.sync_copy(dst_hbm.at[pl.ds(base, ck)], idx_d)
            cp_q = pltpu.async_copy(q_hbm.at[idx_d], pbuf, sem_p)
            pltpu.sync_copy(scores_out.at[pl.ds(base, ck)], sbuf)
            cp_q.wait()

            def group_body(g, carry2):
                e0 = pl.multiple_of(g * _L, _L)
                e_idx = lane + e0
                score = sbuf[pl.ds(e0, _L)]

                def msg_step(k, kv):
                    qv = plsc.load_gather(pbuf, [e_idx, kv])
                    plsc.store_scatter(pbuf, [e_idx, kv], qv * score)
                    return kv + 1

                lax.fori_loop(0, d, msg_step, jnp.zeros((_L,), jnp.int32))
                return carry2

            lax.fori_loop(0, groups, group_body, 0)
            pltpu.sync_copy(pbuf, h_acc.at[idx_s], add=True)
            pltpu.sync_copy(ones_v, deg_acc.at[idx_s], add=True)
            return carry

        lax.fori_loop(0, n_chunks, phase2_chunk, 0)
        plsc.subcore_barrier()
        pltpu.sync_copy(h_acc.at[pl.ds(r0, span)],
                        hout_out.at[pl.ds(out_r0, span)])
        pltpu.sync_copy(deg_acc.at[pl.ds(r0, span)],
                        dout_out.at[pl.ds(out_r0, span)])

    return pl.kernel(
        body,
        out_type=[
            jax.ShapeDtypeStruct((_NC * n_nodes, d), jnp.float32),
            jax.ShapeDtypeStruct((_NC * n_nodes, d), jnp.float32),
            jax.ShapeDtypeStruct((_NC * n_nodes, _DW), jnp.float32),
            jax.ShapeDtypeStruct((_NC * n_nodes, _DW), jnp.float32),
            jax.ShapeDtypeStruct((n_edges,), jnp.float32),
        ],
        mesh=mesh,
        scratch_types=[
            pltpu.VMEM_SHARED((n_nodes, d), jnp.float32),
            pltpu.VMEM_SHARED((n_nodes, _DW), jnp.float32),
            pltpu.VMEM((ck,), jnp.int32),
            pltpu.VMEM((ck,), jnp.int32),
            pltpu.VMEM((ck, d), jnp.float32),
            pltpu.VMEM((ck, d), jnp.float32),
            pltpu.VMEM((ck, d), jnp.float32),
            pltpu.VMEM((ck,), jnp.float32),
            pltpu.VMEM((ck, _DW), jnp.float32),
            pltpu.VMEM((zr, d), jnp.float32),
            pltpu.VMEM((zr, _DW), jnp.float32),
            pltpu.VMEM((2 * d,), jnp.float32),
            pltpu.SemaphoreType.DMA,
            pltpu.SemaphoreType.DMA,
            pltpu.SemaphoreType.DMA,
        ],
        compiler_params=pltpu.CompilerParams(needs_layout_passes=False,
                                             use_tc_tiling_on_sc=False),
        interpret=_INTERPRET,
    )


def kernel(x, edge_index, W_s2d, b_s2d, W_d2s, b_d2s, We1, be1, We2, be2,
           Wg1, bg1, Wg2, bg2):
    n, d = x.shape
    e = edge_index.shape[1]
    src = edge_index[0]
    dst = edge_index[1]

    br = 1000 if n % 1000 == 0 else n
    nb = n // br
    row_spec = pl.BlockSpec((br, d), lambda i: (i, 0))
    full_spec = pl.BlockSpec((d, d), lambda i: (0, 0))
    bias_spec = pl.BlockSpec((1, d), lambda i: (0, 0))
    a_n, b_n, p_n, q_n = pl.pallas_call(
        _pre_body,
        grid=(nb,),
        in_specs=[row_spec, full_spec, full_spec, full_spec, full_spec,
                  bias_spec, bias_spec, bias_spec],
        out_specs=[row_spec] * 4,
        out_shape=[jax.ShapeDtypeStruct((n, d), jnp.float32)] * 4,
        interpret=_INTERPRET,
    )(x, We1[:d], We1[d:], W_s2d, W_d2s,
      be1[None, :], b_s2d[None, :], b_d2s[None, :])

    wb2 = jnp.concatenate([We2[:, 0], be2, jnp.zeros((d - 1,), jnp.float32)])
    ck = 80 if e % (_NC * _NS * 80) == 0 else 16
    zr = 8
    zrow = jnp.zeros((zr, d), jnp.float32)
    zdeg = jnp.zeros((zr, _DW), jnp.float32)
    ones = jnp.ones((ck, _DW), jnp.float32)

    edge_fn = _make_edge_kernel(n, e, d, ck)
    hin_p, hout_p, din_p, dout_p, _ = edge_fn(src, dst, a_n, b_n, p_n, q_n,
                                              wb2, zrow, zdeg, ones)

    lo_spec = pl.BlockSpec((br, d), lambda i: (i, 0))
    hi_spec = pl.BlockSpec((br, d), lambda i: (i + nb, 0))
    dlo_spec = pl.BlockSpec((br, _DW), lambda i: (i, 0))
    dhi_spec = pl.BlockSpec((br, _DW), lambda i: (i + nb, 0))
    out = pl.pallas_call(
        _post_body,
        grid=(nb,),
        in_specs=[row_spec, lo_spec, hi_spec, lo_spec, hi_spec,
                  dlo_spec, dhi_spec, dlo_spec, dhi_spec,
                  full_spec, full_spec, bias_spec, bias_spec,
                  pl.BlockSpec((1, 1), lambda i: (0, 0))],
        out_specs=row_spec,
        out_shape=jax.ShapeDtypeStruct((n, d), jnp.float32),
        interpret=_INTERPRET,
    )(x, hin_p, hin_p, hout_p, hout_p, din_p, din_p, dout_p, dout_p,
      Wg1[:d], Wg1[d:], bg1[None, :], Wg2[:, 0][None, :], bg2[:, None])
    return out


# speedup vs baseline: 1.0181x; 1.0181x over previous
import functools

import jax
import jax.numpy as jnp
from jax import lax
from jax.experimental import pallas as pl
from jax.experimental.pallas import tpu as pltpu
from jax.experimental.pallas import tpu_sc as plsc

_NC = 2
_NS = 16
_L = 16

_INTERPRET = False


def _pre_body(x_ref, wa_ref, wb_ref, wp_ref, wq_ref, bb_ref, bp_ref, bq_ref,
              a_out, b_out, p_out, q_out):
    x = x_ref[...]
    a_out[...] = jnp.dot(x, wa_ref[...], preferred_element_type=jnp.float32)
    b_out[...] = jnp.dot(x, wb_ref[...], preferred_element_type=jnp.float32) + bb_ref[...]
    p_out[...] = jnp.dot(x, wp_ref[...], preferred_element_type=jnp.float32) + bp_ref[...]
    q_out[...] = jnp.dot(x, wq_ref[...], preferred_element_type=jnp.float32) + bq_ref[...]


def _post_body(x_ref, hin0_ref, hin1_ref, hout0_ref, hout1_ref,
               din0_ref, din1_ref, dout0_ref, dout1_ref,
               wga_ref, wgb_ref, bg1_ref, wg2_ref, bg2_ref, out_ref):
    hin = hin0_ref[...] + hin1_ref[...]
    hout = hout0_ref[...] + hout1_ref[...]
    din = jnp.maximum(din0_ref[...] + din1_ref[...], 1.0)
    dout = jnp.maximum(dout0_ref[...] + dout1_ref[...], 1.0)
    h_in = hin / din
    h_out = hout / dout
    gh = jnp.maximum(
        jnp.dot(h_in, wga_ref[...], preferred_element_type=jnp.float32)
        + jnp.dot(h_out, wgb_ref[...], preferred_element_type=jnp.float32)
        + bg1_ref[...], 0.0)
    gz = jnp.sum(gh * wg2_ref[...], axis=1, keepdims=True) + bg2_ref[...]
    g = 1.0 / (1.0 + jnp.exp(-gz))
    out_ref[...] = g * h_in + (1.0 - g) * h_out + x_ref[...]


def _make_edge_kernel(n_nodes, n_edges, d, ck):
    nw = _NC * _NS
    e_per_w = n_edges // nw
    n_chunks = e_per_w // ck
    groups = ck // _L
    assert e_per_w * nw == n_edges and n_chunks * ck == e_per_w
    assert groups * _L == ck and n_nodes % 8 == 0
    span = 8 * (-(-(n_nodes // 8) // _NS))
    zr = 8
    zit = span // zr
    assert zr * zit == span

    mesh = plsc.VectorSubcoreMesh(core_axis_name="c", subcore_axis_name="s",
                                  num_cores=_NC, num_subcores=_NS)

    def body(sd_hbm, a_hbm, b_hbm, p_hbm, q_hbm, wb2_hbm,
             zrow_hbm, zdeg_hbm, ones_hbm,
             hin_out, hout_out, din_out, dout_out, scores_out,
             h_acc, deg_acc,
             idx_a, idx_b, abuf, bbuf, pbuf_a, pbuf_b, sbuf_a, sbuf_b,
             ones_v, zbuf, zdeg, w2_v,
             sem_a, sem_b, sem_p, sem_i, sem_h, sem_g, sem_w):
        c = lax.axis_index("c")
        s = lax.axis_index("s")
        wid = c * _NS + s
        lane = lax.iota(jnp.int32, _L)

        pltpu.sync_copy(wb2_hbm, w2_v)
        pltpu.sync_copy(zrow_hbm, zbuf)
        pltpu.sync_copy(zdeg_hbm, zdeg)
        pltpu.sync_copy(ones_hbm, ones_v)

        r0 = pl.multiple_of(jnp.minimum(s * span, n_nodes - span), 8)

        def clear_acc():
            def zero_step(t, carry):
                off = pl.multiple_of(r0 + t * zr, 8)
                pltpu.sync_copy(zbuf, h_acc.at[pl.ds(off, zr)])
                return carry
            lax.fori_loop(0, zit, zero_step, 0)
            pltpu.sync_copy(zdeg, deg_acc.at[pl.ds(r0, span)])

        clear_acc()
        plsc.subcore_barrier()

        be2v = plsc.load_gather(w2_v, [jnp.full((_L,), d, jnp.int32)])
        kv0 = jnp.zeros((_L,), jnp.int32)

        def compute_scores(src_buf1, src_buf2, out_sbuf):
            def group_body(g, carry2):
                e0 = pl.multiple_of(g * _L, _L)
                e_idx = lane + e0

                def score_step(k, state):
                    acc, kv = state
                    av = plsc.load_gather(src_buf1, [e_idx, kv])
                    bv = plsc.load_gather(src_buf2, [e_idx, kv])
                    wk = plsc.load_gather(w2_v, [kv])
                    return acc + jnp.maximum(av + bv, 0.0) * wk, kv + 1

                acc, _ = lax.fori_loop(0, d, score_step, (be2v, kv0))
                out_sbuf[pl.ds(e0, _L)] = 1.0 / (1.0 + jnp.exp(-acc))
                return carry2

            lax.fori_loop(0, groups, group_body, 0)

        def scale_rows(row_buf, score_buf):
            def group_body(g, carry2):
                e0 = pl.multiple_of(g * _L, _L)
                e_idx = lane + e0
                score = score_buf[pl.ds(e0, _L)]

                def msg_step(k, kv):
                    pv = plsc.load_gather(row_buf, [e_idx, kv])
                    plsc.store_scatter(row_buf, [e_idx, kv], pv * score)
                    return kv + 1

                lax.fori_loop(0, d, msg_step, kv0)
                return carry2

            lax.fori_loop(0, groups, group_body, 0)

        def half1(i, idxc, idxn, pbc, pbn, sbc, sbn):
            @pl.when(i < n_chunks)
            def _run():
                pltpu.make_async_copy(a_hbm.at[idxc.at[0]], abuf, sem_a).wait()
                pltpu.make_async_copy(b_hbm.at[idxc.at[1]], bbuf, sem_b).wait()
                pltpu.make_async_copy(p_hbm.at[idxc.at[0]], pbc, sem_p).wait()

                @pl.when(i > 0)
                def _drain():
                    pltpu.make_async_copy(pbn, h_acc.at[idxn.at[1]], sem_h).wait()
                    pltpu.make_async_copy(ones_v, deg_acc.at[idxn.at[1]], sem_g).wait()
                    pltpu.make_async_copy(sbn, scores_out.at[pl.ds(0, ck)], sem_w).wait()

                @pl.when(i + 1 < n_chunks)
                def _prefetch():
                    pltpu.async_copy(sd_hbm.at[wid * n_chunks + i + 1], idxn, sem_i)

                compute_scores(abuf, bbuf, sbc)
                scale_rows(pbc, sbc)

                @pl.when(i + 1 < n_chunks)
                def _issue_next():
                    pltpu.make_async_copy(sd_hbm.at[wid * n_chunks + i + 1], idxn, sem_i).wait()
                    pltpu.async_copy(a_hbm.at[idxn.at[0]], abuf, sem_a)
                    pltpu.async_copy(b_hbm.at[idxn.at[1]], bbuf, sem_b)
                    pltpu.async_copy(p_hbm.at[idxn.at[0]], pbn, sem_p)

                base = pl.multiple_of((wid * n_chunks + i) * ck, 8)
                pltpu.async_copy(pbc, h_acc.at[idxc.at[1]], sem_h)
                pltpu.async_copy(ones_v, deg_acc.at[idxc.at[1]], sem_g)
                pltpu.async_copy(sbc, scores_out.at[pl.ds(base, ck)], sem_w)

        def serial1(i, carry):
            jid = wid * n_chunks + i
            pltpu.sync_copy(sd_hbm.at[jid], idx_a)
            cpa = pltpu.async_copy(a_hbm.at[idx_a.at[0]], abuf, sem_a)
            cpb = pltpu.async_copy(b_hbm.at[idx_a.at[1]], bbuf, sem_b)
            cpp = pltpu.async_copy(p_hbm.at[idx_a.at[0]], pbuf_a, sem_p)
            cpa.wait(); cpb.wait(); cpp.wait()
            compute_scores(abuf, bbuf, sbuf_a)
            scale_rows(pbuf_a, sbuf_a)
            base = pl.multiple_of(jid * ck, 8)
            pltpu.sync_copy(pbuf_a, h_acc.at[idx_a.at[1]], add=True)
            pltpu.sync_copy(ones_v, deg_acc.at[idx_a.at[1]], add=True)
            pltpu.sync_copy(sbuf_a, scores_out.at[pl.ds(base, ck)])
            return carry

        lax.fori_loop(0, n_chunks, serial1, 0)

        plsc.subcore_barrier()
        out_r0 = pl.multiple_of(c * n_nodes + r0, 8)
        pltpu.sync_copy(h_acc.at[pl.ds(r0, span)],
                        hin_out.at[pl.ds(out_r0, span)])
        pltpu.sync_copy(deg_acc.at[pl.ds(r0, span)],
                        din_out.at[pl.ds(out_r0, span)])
        plsc.subcore_barrier()
        clear_acc()
        plsc.subcore_barrier()

        def half2(i, idxc, idxn, pbc, pbn, sbc, sbn):
            @pl.when(i < n_chunks)
            def _run():
                base = pl.multiple_of((wid * n_chunks + i) * ck, 8)
                pltpu.make_async_copy(q_hbm.at[idxc.at[1]], pbc, sem_p).wait()
                pltpu.make_async_copy(scores_out.at[pl.ds(base, ck)], sbc, sem_w).wait()

                @pl.when(i > 0)
                def _drain():
                    pltpu.make_async_copy(pbn, h_acc.at[idxn.at[0]], sem_h).wait()
                    pltpu.make_async_copy(ones_v, deg_acc.at[idxn.at[0]], sem_g).wait()

                @pl.when(i + 1 < n_chunks)
                def _prefetch():
                    pltpu.async_copy(sd_hbm.at[wid * n_chunks + i + 1], idxn, sem_i)

                scale_rows(pbc, sbc)

                @pl.when(i + 1 < n_chunks)
                def _issue_next():
                    nbase = pl.multiple_of((wid * n_chunks + i + 1) * ck, 8)
                    pltpu.make_async_copy(sd_hbm.at[wid * n_chunks + i + 1], idxn, sem_i).wait()
                    pltpu.async_copy(q_hbm.at[idxn.at[1]], pbn, sem_p)
                    pltpu.async_copy(scores_out.at[pl.ds(nbase, ck)], sbn, sem_w)

                pltpu.async_copy(pbc, h_acc.at[idxc.at[0]], sem_h)
                pltpu.async_copy(ones_v, deg_acc.at[idxc.at[0]], sem_g)

        def serial2(i, carry):
            jid = wid * n_chunks + i
            base = pl.multiple_of(jid * ck, 8)
            pltpu.sync_copy(sd_hbm.at[jid], idx_a)
            cpq = pltpu.async_copy(q_hbm.at[idx_a.at[1]], pbuf_a, sem_p)
            pltpu.sync_copy(scores_out.at[pl.ds(base, ck)], sbuf_a)
            cpq.wait()
            scale_rows(pbuf_a, sbuf_a)
            pltpu.sync_copy(pbuf_a, h_acc.at[idx_a.at[0]], add=True)
            pltpu.sync_copy(ones_v, deg_acc.at[idx_a.at[0]], add=True)
            return carry

        lax.fori_loop(0, n_chunks, serial2, 0)

        plsc.subcore_barrier()
        pltpu.sync_copy(h_acc.at[pl.ds(r0, span)],
                        hout_out.at[pl.ds(out_r0, span)])
        pltpu.sync_copy(deg_acc.at[pl.ds(r0, span)],
                        dout_out.at[pl.ds(out_r0, span)])

    return pl.kernel(
        body,
        out_type=[
            jax.ShapeDtypeStruct((_NC * n_nodes, d), jnp.float32),
            jax.ShapeDtypeStruct((_NC * n_nodes, d), jnp.float32),
            jax.ShapeDtypeStruct((_NC * n_nodes,), jnp.float32),
            jax.ShapeDtypeStruct((_NC * n_nodes,), jnp.float32),
            jax.ShapeDtypeStruct((n_edges,), jnp.float32),
        ],
        mesh=mesh,
        scratch_types=[
            pltpu.VMEM_SHARED((n_nodes, d), jnp.float32),
            pltpu.VMEM_SHARED((n_nodes,), jnp.float32),
            pltpu.VMEM((2, ck), jnp.int32),
            pltpu.VMEM((2, ck), jnp.int32),
            pltpu.VMEM((ck, d), jnp.float32),
            pltpu.VMEM((ck, d), jnp.float32),
            pltpu.VMEM((ck, d), jnp.float32),
            pltpu.VMEM((ck, d), jnp.float32),
            pltpu.VMEM((ck,), jnp.float32),
            pltpu.VMEM((ck,), jnp.float32),
            pltpu.VMEM((ck,), jnp.float32),
            pltpu.VMEM((zr, d), jnp.float32),
            pltpu.VMEM((span,), jnp.float32),
            pltpu.VMEM((2 * d,), jnp.float32),
            pltpu.SemaphoreType.DMA,
            pltpu.SemaphoreType.DMA,
            pltpu.SemaphoreType.DMA,
            pltpu.SemaphoreType.DMA,
            pltpu.SemaphoreType.DMA,
            pltpu.SemaphoreType.DMA,
            pltpu.SemaphoreType.DMA,
        ],
        compiler_params=pltpu.CompilerParams(needs_layout_passes=False,
                                             use_tc_tiling_on_sc=False),
        interpret=_INTERPRET,
    )


def kernel(x, edge_index, W_s2d, b_s2d, W_d2s, b_d2s, We1, be1, We2, be2,
           Wg1, bg1, Wg2, bg2):
    n, d = x.shape
    e = edge_index.shape[1]

    br = 1000 if n % 1000 == 0 else n
    nb = n // br
    row_spec = pl.BlockSpec((br, d), lambda i: (i, 0))
    full_spec = pl.BlockSpec((d, d), lambda i: (0, 0))
    bias_spec = pl.BlockSpec((1, d), lambda i: (0, 0))
    a_n, b_n, p_n, q_n = pl.pallas_call(
        _pre_body,
        grid=(nb,),
        in_specs=[row_spec, full_spec, full_spec, full_spec, full_spec,
                  bias_spec, bias_spec, bias_spec],
        out_specs=[row_spec] * 4,
        out_shape=[jax.ShapeDtypeStruct((n, d), jnp.float32)] * 4,
        interpret=_INTERPRET,
    )(x, We1[:d], We1[d:], W_s2d, W_d2s,
      be1[None, :], b_s2d[None, :], b_d2s[None, :])

    wb2 = jnp.concatenate([We2[:, 0], be2, jnp.zeros((d - 1,), jnp.float32)])
    ck = 80 if e % (_NC * _NS * 80) == 0 else 16
    total_chunks = e // ck
    edge_sd = edge_index.reshape(2, total_chunks, ck).transpose(1, 0, 2)
    span = 8 * (-(-(n // 8) // _NS))
    zr = 8
    zrow = jnp.zeros((zr, d), jnp.float32)
    zdeg = jnp.zeros((span,), jnp.float32)
    ones = jnp.ones((ck,), jnp.float32)

    edge_fn = _make_edge_kernel(n, e, d, ck)
    hin_p, hout_p, din_p, dout_p, _ = edge_fn(edge_sd, a_n, b_n, p_n, q_n,
                                              wb2, zrow, zdeg, ones)

    lo_spec = pl.BlockSpec((br, d), lambda i: (i, 0))
    hi_spec = pl.BlockSpec((br, d), lambda i: (i + nb, 0))
    dlo_spec = pl.BlockSpec((br, 1), lambda i: (i, 0))
    dhi_spec = pl.BlockSpec((br, 1), lambda i: (i + nb, 0))
    din2 = din_p[:, None]
    dout2 = dout_p[:, None]
    out = pl.pallas_call(
        _post_body,
        grid=(nb,),
        in_specs=[row_spec, lo_spec, hi_spec, lo_spec, hi_spec,
                  dlo_spec, dhi_spec, dlo_spec, dhi_spec,
                  full_spec, full_spec, bias_spec, bias_spec,
                  pl.BlockSpec((1, 1), lambda i: (0, 0))],
        out_specs=row_spec,
        out_shape=jax.ShapeDtypeStruct((n, d), jnp.float32),
        interpret=_INTERPRET,
    )(x, hin_p, hin_p, hout_p, hout_p, din2, din2, dout2, dout2,
      Wg1[:d], Wg1[d:], bg1[None, :], Wg2[:, 0][None, :], bg2[:, None])
    return out


# speedup vs baseline: 1.0543x; 1.0355x over previous
import functools

import jax
import jax.numpy as jnp
from jax import lax
from jax.experimental import pallas as pl
from jax.experimental.pallas import tpu as pltpu
from jax.experimental.pallas import tpu_sc as plsc

_NC = 2
_NS = 16
_L = 16

_INTERPRET = False


def _pre_body(x_ref, wa_ref, wb_ref, wp_ref, wq_ref, bb_ref, bp_ref, bq_ref,
              a_out, b_out, p_out, q_out):
    x = x_ref[...]
    a_out[...] = jnp.dot(x, wa_ref[...], preferred_element_type=jnp.float32)
    b_out[...] = jnp.dot(x, wb_ref[...], preferred_element_type=jnp.float32) + bb_ref[...]
    p_out[...] = jnp.dot(x, wp_ref[...], preferred_element_type=jnp.float32) + bp_ref[...]
    q_out[...] = jnp.dot(x, wq_ref[...], preferred_element_type=jnp.float32) + bq_ref[...]


def _post_body(x_ref, hin0_ref, hin1_ref, hout0_ref, hout1_ref,
               din0_ref, din1_ref, dout0_ref, dout1_ref,
               wga_ref, wgb_ref, bg1_ref, wg2_ref, bg2_ref, out_ref):
    hin = hin0_ref[...] + hin1_ref[...]
    hout = hout0_ref[...] + hout1_ref[...]
    din = jnp.maximum(din0_ref[...] + din1_ref[...], 1.0)
    dout = jnp.maximum(dout0_ref[...] + dout1_ref[...], 1.0)
    h_in = hin / din
    h_out = hout / dout
    gh = jnp.maximum(
        jnp.dot(h_in, wga_ref[...], preferred_element_type=jnp.float32)
        + jnp.dot(h_out, wgb_ref[...], preferred_element_type=jnp.float32)
        + bg1_ref[...], 0.0)
    gz = jnp.sum(gh * wg2_ref[...], axis=1, keepdims=True) + bg2_ref[...]
    g = 1.0 / (1.0 + jnp.exp(-gz))
    out_ref[...] = g * h_in + (1.0 - g) * h_out + x_ref[...]


def _make_edge_kernel(n_nodes, n_edges, d, ck):
    nw = _NC * _NS
    e_per_w = n_edges // nw
    n_chunks = e_per_w // ck
    groups = ck // _L
    assert e_per_w * nw == n_edges and n_chunks * ck == e_per_w
    assert groups * _L == ck and n_nodes % 8 == 0
    span = 8 * (-(-(n_nodes // 8) // _NS))
    zr = 8
    zit = span // zr
    assert zr * zit == span

    mesh = plsc.VectorSubcoreMesh(core_axis_name="c", subcore_axis_name="s",
                                  num_cores=_NC, num_subcores=_NS)

    def body(sd_hbm, a_hbm, b_hbm, p_hbm, q_hbm, wb2_hbm,
             zrow_hbm, zdeg_hbm, ones_hbm,
             hin_out, hout_out, din_out, dout_out, scores_out,
             h_acc, deg_acc,
             idx_a, idx_b, abuf, bbuf, pbuf_a, pbuf_b, sbuf_a, sbuf_b,
             ones_v, zbuf, zdeg, w2_v,
             sem_a, sem_b, sem_p, sem_i, sem_h, sem_g, sem_w):
        c = lax.axis_index("c")
        s = lax.axis_index("s")
        wid = c * _NS + s
        lane = lax.iota(jnp.int32, _L)

        pltpu.sync_copy(wb2_hbm, w2_v)
        pltpu.sync_copy(zrow_hbm, zbuf)
        pltpu.sync_copy(zdeg_hbm, zdeg)
        pltpu.sync_copy(ones_hbm, ones_v)

        r0 = pl.multiple_of(jnp.minimum(s * span, n_nodes - span), 8)

        def clear_acc():
            def zero_step(t, carry):
                off = pl.multiple_of(r0 + t * zr, 8)
                pltpu.sync_copy(zbuf, h_acc.at[pl.ds(off, zr)])
                return carry
            lax.fori_loop(0, zit, zero_step, 0)
            pltpu.sync_copy(zdeg, deg_acc.at[pl.ds(r0, span)])

        clear_acc()
        plsc.subcore_barrier()

        be2v = plsc.load_gather(w2_v, [jnp.full((_L,), d, jnp.int32)])
        kv0 = jnp.zeros((_L,), jnp.int32)

        unroll = 8
        assert d % unroll == 0

        def compute_scores(src_buf1, src_buf2, out_sbuf):
            def group_body(g, carry2):
                e0 = pl.multiple_of(g * _L, _L)
                e_idx = lane + e0

                def score_step(k, state):
                    acc, kv = state
                    for u in range(unroll):
                        kvu = kv + u
                        av = plsc.load_gather(src_buf1, [e_idx, kvu])
                        bv = plsc.load_gather(src_buf2, [e_idx, kvu])
                        wk = plsc.load_gather(w2_v, [kvu])
                        acc = acc + jnp.maximum(av + bv, 0.0) * wk
                    return acc, kv + unroll

                acc, _ = lax.fori_loop(0, d // unroll, score_step, (be2v, kv0))
                out_sbuf[pl.ds(e0, _L)] = 1.0 / (1.0 + jnp.exp(-acc))
                return carry2

            lax.fori_loop(0, groups, group_body, 0)

        def scale_rows(row_buf, score_buf):
            def group_body(g, carry2):
                e0 = pl.multiple_of(g * _L, _L)
                e_idx = lane + e0
                score = score_buf[pl.ds(e0, _L)]

                def msg_step(k, kv):
                    for u in range(unroll):
                        kvu = kv + u
                        pv = plsc.load_gather(row_buf, [e_idx, kvu])
                        plsc.store_scatter(row_buf, [e_idx, kvu], pv * score)
                    return kv + unroll

                lax.fori_loop(0, d // unroll, msg_step, kv0)
                return carry2

            lax.fori_loop(0, groups, group_body, 0)

        def half1(i, idxc, idxn, pbc, pbn, sbc, sbn):
            @pl.when(i < n_chunks)
            def _run():
                pltpu.make_async_copy(a_hbm.at[idxc.at[0]], abuf, sem_a).wait()
                pltpu.make_async_copy(b_hbm.at[idxc.at[1]], bbuf, sem_b).wait()
                pltpu.make_async_copy(p_hbm.at[idxc.at[0]], pbc, sem_p).wait()

                @pl.when(i > 0)
                def _drain():
                    pltpu.make_async_copy(pbn, h_acc.at[idxn.at[1]], sem_h).wait()
                    pltpu.make_async_copy(ones_v, deg_acc.at[idxn.at[1]], sem_g).wait()
                    pltpu.make_async_copy(sbn, scores_out.at[pl.ds(0, ck)], sem_w).wait()

                @pl.when(i + 1 < n_chunks)
                def _prefetch():
                    pltpu.async_copy(sd_hbm.at[wid * n_chunks + i + 1], idxn, sem_i)

                compute_scores(abuf, bbuf, sbc)
                scale_rows(pbc, sbc)

                @pl.when(i + 1 < n_chunks)
                def _issue_next():
                    pltpu.make_async_copy(sd_hbm.at[wid * n_chunks + i + 1], idxn, sem_i).wait()
                    pltpu.async_copy(a_hbm.at[idxn.at[0]], abuf, sem_a)
                    pltpu.async_copy(b_hbm.at[idxn.at[1]], bbuf, sem_b)
                    pltpu.async_copy(p_hbm.at[idxn.at[0]], pbn, sem_p)

                base = pl.multiple_of((wid * n_chunks + i) * ck, 8)
                pltpu.async_copy(pbc, h_acc.at[idxc.at[1]], sem_h)
                pltpu.async_copy(ones_v, deg_acc.at[idxc.at[1]], sem_g)
                pltpu.async_copy(sbc, scores_out.at[pl.ds(base, ck)], sem_w)

        def serial1(i, carry):
            jid = wid * n_chunks + i
            pltpu.sync_copy(sd_hbm.at[jid], idx_a)
            cpa = pltpu.async_copy(a_hbm.at[idx_a.at[0]], abuf, sem_a)
            cpb = pltpu.async_copy(b_hbm.at[idx_a.at[1]], bbuf, sem_b)
            cpp = pltpu.async_copy(p_hbm.at[idx_a.at[0]], pbuf_a, sem_p)
            cpa.wait(); cpb.wait(); cpp.wait()
            compute_scores(abuf, bbuf, sbuf_a)
            scale_rows(pbuf_a, sbuf_a)
            base = pl.multiple_of(jid * ck, 8)
            pltpu.sync_copy(pbuf_a, h_acc.at[idx_a.at[1]], add=True)
            pltpu.sync_copy(ones_v, deg_acc.at[idx_a.at[1]], add=True)
            pltpu.sync_copy(sbuf_a, scores_out.at[pl.ds(base, ck)])
            return carry

        lax.fori_loop(0, n_chunks, serial1, 0)

        plsc.subcore_barrier()
        out_r0 = pl.multiple_of(c * n_nodes + r0, 8)
        pltpu.sync_copy(h_acc.at[pl.ds(r0, span)],
                        hin_out.at[pl.ds(out_r0, span)])
        pltpu.sync_copy(deg_acc.at[pl.ds(r0, span)],
                        din_out.at[pl.ds(out_r0, span)])
        plsc.subcore_barrier()
        clear_acc()
        plsc.subcore_barrier()

        def half2(i, idxc, idxn, pbc, pbn, sbc, sbn):
            @pl.when(i < n_chunks)
            def _run():
                base = pl.multiple_of((wid * n_chunks + i) * ck, 8)
                pltpu.make_async_copy(q_hbm.at[idxc.at[1]], pbc, sem_p).wait()
                pltpu.make_async_copy(scores_out.at[pl.ds(base, ck)], sbc, sem_w).wait()

                @pl.when(i > 0)
                def _drain():
                    pltpu.make_async_copy(pbn, h_acc.at[idxn.at[0]], sem_h).wait()
                    pltpu.make_async_copy(ones_v, deg_acc.at[idxn.at[0]], sem_g).wait()

                @pl.when(i + 1 < n_chunks)
                def _prefetch():
                    pltpu.async_copy(sd_hbm.at[wid * n_chunks + i + 1], idxn, sem_i)

                scale_rows(pbc, sbc)

                @pl.when(i + 1 < n_chunks)
                def _issue_next():
                    nbase = pl.multiple_of((wid * n_chunks + i + 1) * ck, 8)
                    pltpu.make_async_copy(sd_hbm.at[wid * n_chunks + i + 1], idxn, sem_i).wait()
                    pltpu.async_copy(q_hbm.at[idxn.at[1]], pbn, sem_p)
                    pltpu.async_copy(scores_out.at[pl.ds(nbase, ck)], sbn, sem_w)

                pltpu.async_copy(pbc, h_acc.at[idxc.at[0]], sem_h)
                pltpu.async_copy(ones_v, deg_acc.at[idxc.at[0]], sem_g)

        def serial2(i, carry):
            jid = wid * n_chunks + i
            base = pl.multiple_of(jid * ck, 8)
            pltpu.sync_copy(sd_hbm.at[jid], idx_a)
            cpq = pltpu.async_copy(q_hbm.at[idx_a.at[1]], pbuf_a, sem_p)
            pltpu.sync_copy(scores_out.at[pl.ds(base, ck)], sbuf_a)
            cpq.wait()
            scale_rows(pbuf_a, sbuf_a)
            pltpu.sync_copy(pbuf_a, h_acc.at[idx_a.at[0]], add=True)
            pltpu.sync_copy(ones_v, deg_acc.at[idx_a.at[0]], add=True)
            return carry

        lax.fori_loop(0, n_chunks, serial2, 0)

        plsc.subcore_barrier()
        pltpu.sync_copy(h_acc.at[pl.ds(r0, span)],
                        hout_out.at[pl.ds(out_r0, span)])
        pltpu.sync_copy(deg_acc.at[pl.ds(r0, span)],
                        dout_out.at[pl.ds(out_r0, span)])

    return pl.kernel(
        body,
        out_type=[
            jax.ShapeDtypeStruct((_NC * n_nodes, d), jnp.float32),
            jax.ShapeDtypeStruct((_NC * n_nodes, d), jnp.float32),
            jax.ShapeDtypeStruct((_NC * n_nodes,), jnp.float32),
            jax.ShapeDtypeStruct((_NC * n_nodes,), jnp.float32),
            jax.ShapeDtypeStruct((n_edges,), jnp.float32),
        ],
        mesh=mesh,
        scratch_types=[
            pltpu.VMEM_SHARED((n_nodes, d), jnp.float32),
            pltpu.VMEM_SHARED((n_nodes,), jnp.float32),
            pltpu.VMEM((2, ck), jnp.int32),
            pltpu.VMEM((2, ck), jnp.int32),
            pltpu.VMEM((ck, d), jnp.float32),
            pltpu.VMEM((ck, d), jnp.float32),
            pltpu.VMEM((ck, d), jnp.float32),
            pltpu.VMEM((ck, d), jnp.float32),
            pltpu.VMEM((ck,), jnp.float32),
            pltpu.VMEM((ck,), jnp.float32),
            pltpu.VMEM((ck,), jnp.float32),
            pltpu.VMEM((zr, d), jnp.float32),
            pltpu.VMEM((span,), jnp.float32),
            pltpu.VMEM((2 * d,), jnp.float32),
            pltpu.SemaphoreType.DMA,
            pltpu.SemaphoreType.DMA,
            pltpu.SemaphoreType.DMA,
            pltpu.SemaphoreType.DMA,
            pltpu.SemaphoreType.DMA,
            pltpu.SemaphoreType.DMA,
            pltpu.SemaphoreType.DMA,
        ],
        compiler_params=pltpu.CompilerParams(needs_layout_passes=False,
                                             use_tc_tiling_on_sc=False),
        interpret=_INTERPRET,
    )


def kernel(x, edge_index, W_s2d, b_s2d, W_d2s, b_d2s, We1, be1, We2, be2,
           Wg1, bg1, Wg2, bg2):
    n, d = x.shape
    e = edge_index.shape[1]

    br = 1000 if n % 1000 == 0 else n
    nb = n // br
    row_spec = pl.BlockSpec((br, d), lambda i: (i, 0))
    full_spec = pl.BlockSpec((d, d), lambda i: (0, 0))
    bias_spec = pl.BlockSpec((1, d), lambda i: (0, 0))
    a_n, b_n, p_n, q_n = pl.pallas_call(
        _pre_body,
        grid=(nb,),
        in_specs=[row_spec, full_spec, full_spec, full_spec, full_spec,
                  bias_spec, bias_spec, bias_spec],
        out_specs=[row_spec] * 4,
        out_shape=[jax.ShapeDtypeStruct((n, d), jnp.float32)] * 4,
        interpret=_INTERPRET,
    )(x, We1[:d], We1[d:], W_s2d, W_d2s,
      be1[None, :], b_s2d[None, :], b_d2s[None, :])

    wb2 = jnp.concatenate([We2[:, 0], be2, jnp.zeros((d - 1,), jnp.float32)])
    ck = 80 if e % (_NC * _NS * 80) == 0 else 16
    total_chunks = e // ck
    edge_sd = edge_index.reshape(2, total_chunks, ck).transpose(1, 0, 2)
    span = 8 * (-(-(n // 8) // _NS))
    zr = 8
    zrow = jnp.zeros((zr, d), jnp.float32)
    zdeg = jnp.zeros((span,), jnp.float32)
    ones = jnp.ones((ck,), jnp.float32)

    edge_fn = _make_edge_kernel(n, e, d, ck)
    hin_p, hout_p, din_p, dout_p, _ = edge_fn(edge_sd, a_n, b_n, p_n, q_n,
                                              wb2, zrow, zdeg, ones)

    lo_spec = pl.BlockSpec((br, d), lambda i: (i, 0))
    hi_spec = pl.BlockSpec((br, d), lambda i: (i + nb, 0))
    dlo_spec = pl.BlockSpec((br, 1), lambda i: (i, 0))
    dhi_spec = pl.BlockSpec((br, 1), lambda i: (i + nb, 0))
    din2 = din_p[:, None]
    dout2 = dout_p[:, None]
    out = pl.pallas_call(
        _post_body,
        grid=(nb,),
        in_specs=[row_spec, lo_spec, hi_spec, lo_spec, hi_spec,
                  dlo_spec, dhi_spec, dlo_spec, dhi_spec,
                  full_spec, full_spec, bias_spec, bias_spec,
                  pl.BlockSpec((1, 1), lambda i: (0, 0))],
        out_specs=row_spec,
        out_shape=jax.ShapeDtypeStruct((n, d), jnp.float32),
        interpret=_INTERPRET,
    )(x, hin_p, hin_p, hout_p, hout_p, din2, din2, dout2, dout2,
      Wg1[:d], Wg1[d:], bg1[None, :], Wg2[:, 0][None, :], bg2[:, None])
    return out


# speedup vs baseline: 4.8821x; 4.6306x over previous
import functools

import jax
import jax.numpy as jnp
from jax import lax
from jax.experimental import pallas as pl
from jax.experimental.pallas import tpu as pltpu
from jax.experimental.pallas import tpu_sc as plsc

_NC = 2
_NS = 16
_L = 16

_INTERPRET = False


def _pre_body(x_ref, wa_ref, wb_ref, wp_ref, wq_ref, bb_ref, bp_ref, bq_ref,
              a_out, b_out, p_out, q_out):
    x = x_ref[...]
    a_out[...] = jnp.dot(x, wa_ref[...], preferred_element_type=jnp.float32)
    b_out[...] = jnp.dot(x, wb_ref[...], preferred_element_type=jnp.float32) + bb_ref[...]
    p_out[...] = jnp.dot(x, wp_ref[...], preferred_element_type=jnp.float32) + bp_ref[...]
    q_out[...] = jnp.dot(x, wq_ref[...], preferred_element_type=jnp.float32) + bq_ref[...]


def _post_body(x_ref, hin0_ref, hin1_ref, hout0_ref, hout1_ref,
               din0_ref, din1_ref, dout0_ref, dout1_ref,
               wga_ref, wgb_ref, bg1_ref, wg2_ref, bg2_ref, out_ref):
    hin = hin0_ref[...] + hin1_ref[...]
    hout = hout0_ref[...] + hout1_ref[...]
    din = jnp.maximum(din0_ref[...] + din1_ref[...], 1.0)
    dout = jnp.maximum(dout0_ref[...] + dout1_ref[...], 1.0)
    h_in = hin / din
    h_out = hout / dout
    gh = jnp.maximum(
        jnp.dot(h_in, wga_ref[...], preferred_element_type=jnp.float32)
        + jnp.dot(h_out, wgb_ref[...], preferred_element_type=jnp.float32)
        + bg1_ref[...], 0.0)
    gz = jnp.sum(gh * wg2_ref[...], axis=1, keepdims=True) + bg2_ref[...]
    g = 1.0 / (1.0 + jnp.exp(-gz))
    out_ref[...] = g * h_in + (1.0 - g) * h_out + x_ref[...]


def _make_edge_kernel(n_nodes, n_edges, d, ck):
    nw = _NC * _NS
    e_per_w = n_edges // nw
    n_chunks = e_per_w // ck
    groups = ck // _L
    assert e_per_w * nw == n_edges and n_chunks * ck == e_per_w
    assert groups * _L == ck and n_nodes % 8 == 0
    span = 8 * (-(-(n_nodes // 8) // _NS))
    zr = 8
    zit = span // zr
    assert zr * zit == span

    mesh = plsc.VectorSubcoreMesh(core_axis_name="c", subcore_axis_name="s",
                                  num_cores=_NC, num_subcores=_NS)

    def body(sd_hbm, a_hbm, b_hbm, p_hbm, q_hbm, wb2_hbm,
             zrow_hbm, zdeg_hbm, ones_hbm,
             hin_out, hout_out, din_out, dout_out, scores_out,
             h_acc, deg_acc,
             idx_a, idx_b, abuf, bbuf, pbuf_a, pbuf_b, sbuf_a, sbuf_b,
             ones_v, zbuf, zdeg, w2_v,
             sem_a, sem_b, sem_p, sem_i, sem_h, sem_g, sem_w):
        c = lax.axis_index("c")
        s = lax.axis_index("s")
        wid = c * _NS + s
        lane = lax.iota(jnp.int32, _L)

        pltpu.sync_copy(wb2_hbm, w2_v)
        pltpu.sync_copy(zrow_hbm, zbuf)
        pltpu.sync_copy(zdeg_hbm, zdeg)
        pltpu.sync_copy(ones_hbm, ones_v)

        r0 = pl.multiple_of(jnp.minimum(s * span, n_nodes - span), 8)

        def clear_acc():
            def zero_step(t, carry):
                off = pl.multiple_of(r0 + t * zr, 8)
                pltpu.sync_copy(zbuf, h_acc.at[pl.ds(off, zr)])
                return carry
            lax.fori_loop(0, zit, zero_step, 0)
            pltpu.sync_copy(zdeg, deg_acc.at[pl.ds(r0, span)])

        clear_acc()
        plsc.subcore_barrier()

        w_slices = [w2_v[pl.ds(j * _L, _L)] for j in range(d // _L)]
        be2s = w2_v[pl.ds(d, _L)][0]
        zero16 = jnp.zeros((_L,), jnp.float32)

        def compute_scores(src_buf1, src_buf2, out_sbuf):
            def group_step(g, carry2):
                e0 = pl.multiple_of(g * _L, _L)
                sv = zero16
                for u in range(_L):
                    e = e0 + u
                    acc = zero16
                    for j in range(d // _L):
                        va = src_buf1[e, pl.ds(j * _L, _L)]
                        vb = src_buf2[e, pl.ds(j * _L, _L)]
                        acc = acc + jnp.maximum(va + vb, 0.0) * w_slices[j]
                    z = jnp.sum(acc) + be2s
                    sv = jnp.where(lane == u, jnp.full((_L,), z, jnp.float32), sv)
                out_sbuf[pl.ds(e0, _L)] = 1.0 / (1.0 + jnp.exp(-sv))
                return carry2

            lax.fori_loop(0, groups, group_step, 0)

        def scale_rows(row_buf, score_buf):
            def group_step(g, carry2):
                e0 = pl.multiple_of(g * _L, _L)
                zv = score_buf[pl.ds(e0, _L)]
                for u in range(_L):
                    e = e0 + u
                    sv = jnp.full((_L,), zv[u], jnp.float32)
                    for j in range(d // _L):
                        sl = pl.ds(j * _L, _L)
                        row_buf[e, sl] = row_buf[e, sl] * sv
                return carry2

            lax.fori_loop(0, groups, group_step, 0)

        def half1(i, idxc, idxn, pbc, pbn, sbc, sbn):
            @pl.when(i < n_chunks)
            def _run():
                pltpu.make_async_copy(a_hbm.at[idxc.at[0]], abuf, sem_a).wait()
                pltpu.make_async_copy(b_hbm.at[idxc.at[1]], bbuf, sem_b).wait()
                pltpu.make_async_copy(p_hbm.at[idxc.at[0]], pbc, sem_p).wait()

                @pl.when(i > 0)
                def _drain():
                    pltpu.make_async_copy(pbn, h_acc.at[idxn.at[1]], sem_h).wait()
                    pltpu.make_async_copy(ones_v, deg_acc.at[idxn.at[1]], sem_g).wait()
                    pltpu.make_async_copy(sbn, scores_out.at[pl.ds(0, ck)], sem_w).wait()

                @pl.when(i + 1 < n_chunks)
                def _prefetch():
                    pltpu.async_copy(sd_hbm.at[wid * n_chunks + i + 1], idxn, sem_i)

                compute_scores(abuf, bbuf, sbc)
                scale_rows(pbc, sbc)

                @pl.when(i + 1 < n_chunks)
                def _issue_next():
                    pltpu.make_async_copy(sd_hbm.at[wid * n_chunks + i + 1], idxn, sem_i).wait()
                    pltpu.async_copy(a_hbm.at[idxn.at[0]], abuf, sem_a)
                    pltpu.async_copy(b_hbm.at[idxn.at[1]], bbuf, sem_b)
                    pltpu.async_copy(p_hbm.at[idxn.at[0]], pbn, sem_p)

                base = pl.multiple_of((wid * n_chunks + i) * ck, 8)
                pltpu.async_copy(pbc, h_acc.at[idxc.at[1]], sem_h)
                pltpu.async_copy(ones_v, deg_acc.at[idxc.at[1]], sem_g)
                pltpu.async_copy(sbc, scores_out.at[pl.ds(base, ck)], sem_w)

        def serial1(i, carry):
            jid = wid * n_chunks + i
            pltpu.sync_copy(sd_hbm.at[jid], idx_a)
            cpa = pltpu.async_copy(a_hbm.at[idx_a.at[0]], abuf, sem_a)
            cpb = pltpu.async_copy(b_hbm.at[idx_a.at[1]], bbuf, sem_b)
            cpp = pltpu.async_copy(p_hbm.at[idx_a.at[0]], pbuf_a, sem_p)
            cpa.wait(); cpb.wait(); cpp.wait()
            compute_scores(abuf, bbuf, sbuf_a)
            scale_rows(pbuf_a, sbuf_a)
            base = pl.multiple_of(jid * ck, 8)
            pltpu.sync_copy(pbuf_a, h_acc.at[idx_a.at[1]], add=True)
            pltpu.sync_copy(ones_v, deg_acc.at[idx_a.at[1]], add=True)
            pltpu.sync_copy(sbuf_a, scores_out.at[pl.ds(base, ck)])
            return carry

        lax.fori_loop(0, n_chunks, serial1, 0)

        plsc.subcore_barrier()
        out_r0 = pl.multiple_of(c * n_nodes + r0, 8)
        pltpu.sync_copy(h_acc.at[pl.ds(r0, span)],
                        hin_out.at[pl.ds(out_r0, span)])
        pltpu.sync_copy(deg_acc.at[pl.ds(r0, span)],
                        din_out.at[pl.ds(out_r0, span)])
        plsc.subcore_barrier()
        clear_acc()
        plsc.subcore_barrier()

        def half2(i, idxc, idxn, pbc, pbn, sbc, sbn):
            @pl.when(i < n_chunks)
            def _run():
                base = pl.multiple_of((wid * n_chunks + i) * ck, 8)
                pltpu.make_async_copy(q_hbm.at[idxc.at[1]], pbc, sem_p).wait()
                pltpu.make_async_copy(scores_out.at[pl.ds(base, ck)], sbc, sem_w).wait()

                @pl.when(i > 0)
                def _drain():
                    pltpu.make_async_copy(pbn, h_acc.at[idxn.at[0]], sem_h).wait()
                    pltpu.make_async_copy(ones_v, deg_acc.at[idxn.at[0]], sem_g).wait()

                @pl.when(i + 1 < n_chunks)
                def _prefetch():
                    pltpu.async_copy(sd_hbm.at[wid * n_chunks + i + 1], idxn, sem_i)

                scale_rows(pbc, sbc)

                @pl.when(i + 1 < n_chunks)
                def _issue_next():
                    nbase = pl.multiple_of((wid * n_chunks + i + 1) * ck, 8)
                    pltpu.make_async_copy(sd_hbm.at[wid * n_chunks + i + 1], idxn, sem_i).wait()
                    pltpu.async_copy(q_hbm.at[idxn.at[1]], pbn, sem_p)
                    pltpu.async_copy(scores_out.at[pl.ds(nbase, ck)], sbn, sem_w)

                pltpu.async_copy(pbc, h_acc.at[idxc.at[0]], sem_h)
                pltpu.async_copy(ones_v, deg_acc.at[idxc.at[0]], sem_g)

        def serial2(i, carry):
            jid = wid * n_chunks + i
            base = pl.multiple_of(jid * ck, 8)
            pltpu.sync_copy(sd_hbm.at[jid], idx_a)
            cpq = pltpu.async_copy(q_hbm.at[idx_a.at[1]], pbuf_a, sem_p)
            pltpu.sync_copy(scores_out.at[pl.ds(base, ck)], sbuf_a)
            cpq.wait()
            scale_rows(pbuf_a, sbuf_a)
            pltpu.sync_copy(pbuf_a, h_acc.at[idx_a.at[0]], add=True)
            pltpu.sync_copy(ones_v, deg_acc.at[idx_a.at[0]], add=True)
            return carry

        lax.fori_loop(0, n_chunks, serial2, 0)

        plsc.subcore_barrier()
        pltpu.sync_copy(h_acc.at[pl.ds(r0, span)],
                        hout_out.at[pl.ds(out_r0, span)])
        pltpu.sync_copy(deg_acc.at[pl.ds(r0, span)],
                        dout_out.at[pl.ds(out_r0, span)])

    return pl.kernel(
        body,
        out_type=[
            jax.ShapeDtypeStruct((_NC * n_nodes, d), jnp.float32),
            jax.ShapeDtypeStruct((_NC * n_nodes, d), jnp.float32),
            jax.ShapeDtypeStruct((_NC * n_nodes,), jnp.float32),
            jax.ShapeDtypeStruct((_NC * n_nodes,), jnp.float32),
            jax.ShapeDtypeStruct((n_edges,), jnp.float32),
        ],
        mesh=mesh,
        scratch_types=[
            pltpu.VMEM_SHARED((n_nodes, d), jnp.float32),
            pltpu.VMEM_SHARED((n_nodes,), jnp.float32),
            pltpu.VMEM((2, ck), jnp.int32),
            pltpu.VMEM((2, ck), jnp.int32),
            pltpu.VMEM((ck, d), jnp.float32),
            pltpu.VMEM((ck, d), jnp.float32),
            pltpu.VMEM((ck, d), jnp.float32),
            pltpu.VMEM((ck, d), jnp.float32),
            pltpu.VMEM((ck,), jnp.float32),
            pltpu.VMEM((ck,), jnp.float32),
            pltpu.VMEM((ck,), jnp.float32),
            pltpu.VMEM((zr, d), jnp.float32),
            pltpu.VMEM((span,), jnp.float32),
            pltpu.VMEM((2 * d,), jnp.float32),
            pltpu.SemaphoreType.DMA,
            pltpu.SemaphoreType.DMA,
            pltpu.SemaphoreType.DMA,
            pltpu.SemaphoreType.DMA,
            pltpu.SemaphoreType.DMA,
            pltpu.SemaphoreType.DMA,
            pltpu.SemaphoreType.DMA,
        ],
        compiler_params=pltpu.CompilerParams(needs_layout_passes=False,
                                             use_tc_tiling_on_sc=False),
        interpret=_INTERPRET,
    )


def kernel(x, edge_index, W_s2d, b_s2d, W_d2s, b_d2s, We1, be1, We2, be2,
           Wg1, bg1, Wg2, bg2):
    n, d = x.shape
    e = edge_index.shape[1]

    br = 1000 if n % 1000 == 0 else n
    nb = n // br
    row_spec = pl.BlockSpec((br, d), lambda i: (i, 0))
    full_spec = pl.BlockSpec((d, d), lambda i: (0, 0))
    bias_spec = pl.BlockSpec((1, d), lambda i: (0, 0))
    a_n, b_n, p_n, q_n = pl.pallas_call(
        _pre_body,
        grid=(nb,),
        in_specs=[row_spec, full_spec, full_spec, full_spec, full_spec,
                  bias_spec, bias_spec, bias_spec],
        out_specs=[row_spec] * 4,
        out_shape=[jax.ShapeDtypeStruct((n, d), jnp.float32)] * 4,
        interpret=_INTERPRET,
    )(x, We1[:d], We1[d:], W_s2d, W_d2s,
      be1[None, :], b_s2d[None, :], b_d2s[None, :])

    wb2 = jnp.concatenate([We2[:, 0], be2, jnp.zeros((d - 1,), jnp.float32)])
    ck = 80 if e % (_NC * _NS * 80) == 0 else 16
    total_chunks = e // ck
    edge_sd = edge_index.reshape(2, total_chunks, ck).transpose(1, 0, 2)
    span = 8 * (-(-(n // 8) // _NS))
    zr = 8
    zrow = jnp.zeros((zr, d), jnp.float32)
    zdeg = jnp.zeros((span,), jnp.float32)
    ones = jnp.ones((ck,), jnp.float32)

    edge_fn = _make_edge_kernel(n, e, d, ck)
    hin_p, hout_p, din_p, dout_p, _ = edge_fn(edge_sd, a_n, b_n, p_n, q_n,
                                              wb2, zrow, zdeg, ones)

    lo_spec = pl.BlockSpec((br, d), lambda i: (i, 0))
    hi_spec = pl.BlockSpec((br, d), lambda i: (i + nb, 0))
    dlo_spec = pl.BlockSpec((br, 1), lambda i: (i, 0))
    dhi_spec = pl.BlockSpec((br, 1), lambda i: (i + nb, 0))
    din2 = din_p[:, None]
    dout2 = dout_p[:, None]
    out = pl.pallas_call(
        _post_body,
        grid=(nb,),
        in_specs=[row_spec, lo_spec, hi_spec, lo_spec, hi_spec,
                  dlo_spec, dhi_spec, dlo_spec, dhi_spec,
                  full_spec, full_spec, bias_spec, bias_spec,
                  pl.BlockSpec((1, 1), lambda i: (0, 0))],
        out_specs=row_spec,
        out_shape=jax.ShapeDtypeStruct((n, d), jnp.float32),
        interpret=_INTERPRET,
    )(x, hin_p, hin_p, hout_p, hout_p, din2, din2, dout2, dout2,
      Wg1[:d], Wg1[d:], bg1[None, :], Wg2[:, 0][None, :], bg2[:, None])
    return out
